# Initial kernel scaffold; baseline (speedup 1.0000x reference)
import jax
import jax.numpy as jnp
from jax.experimental import pallas as pl


def kernel(source, target, times, h_s, h_s_times, h_s_mask, nt, embeddings, delta_table):
    raise NotImplementedError("write your pallas kernel here")


# speedup vs baseline: 3.9163x; 3.9163x over previous
import functools

import jax
import jax.numpy as jnp
from jax import lax
from jax.experimental import pallas as pl
from jax.experimental.pallas import tpu as pltpu
from jax.experimental.pallas import tpu_sc as plsc

NODE = 100000
D = 128
B = 16384
H = 20
NEG = 20

NC = 2
NS = 16
NW = NC * NS
BPW = B // NW
CH = 8
NCHUNK = BPW // CH
HHALF = CH * H // 2


def _sc_scores(source, target, h_s_flat, nt, embeddings, delta1d):
    mesh = plsc.VectorSubcoreMesh(
        core_axis_name="c", subcore_axis_name="s",
        num_cores=NC, num_subcores=NS)

    @functools.partial(
        pl.kernel,
        out_type=(jax.ShapeDtypeStruct((B, 32), jnp.float32),
                  jax.ShapeDtypeStruct((B, 32), jnp.float32)),
        mesh=mesh,
        compiler_params=pltpu.CompilerParams(needs_layout_passes=False),
        scratch_types=[
            pltpu.VMEM((NEG,), jnp.int32),
            pltpu.VMEM((NEG, D), jnp.float32),
            pltpu.VMEM((CH,), jnp.int32),
            pltpu.VMEM((CH,), jnp.int32),
            pltpu.VMEM((HHALF,), jnp.int32),
            pltpu.VMEM((HHALF,), jnp.int32),
            pltpu.VMEM((CH, D), jnp.float32),
            pltpu.VMEM((CH, D), jnp.float32),
            pltpu.VMEM((CH * H, D), jnp.float32),
            pltpu.VMEM((16,), jnp.float32),
            pltpu.VMEM((CH, 32), jnp.float32),
            pltpu.VMEM((CH, 32), jnp.float32),
            pltpu.SemaphoreType.DMA,
        ],
    )
    def k(src_h, tgt_h, hs_h, nt_h, emb_h, dlt_h, a_out, n_out,
          nt_idx, neg_rows, s_idx, t_idx, h_idx_a, h_idx_b,
          s_rows, t_rows, h_rows, dlt_v, a_v, n_v, sem):
        wid = lax.axis_index("s") * NC + lax.axis_index("c")
        base = wid * BPW
        pltpu.sync_copy(nt_h, nt_idx)
        pltpu.async_copy(emb_h.at[nt_idx], neg_rows, sem).wait()
        lanes = lax.iota(jnp.int32, 16)

        def chunk_body(c, carry):
            off = base + c * CH
            pltpu.sync_copy(src_h.at[pl.ds(off, CH)], s_idx)
            pltpu.sync_copy(tgt_h.at[pl.ds(off, CH)], t_idx)
            hoff = off * H
            pltpu.sync_copy(hs_h.at[pl.ds(hoff, HHALF)], h_idx_a)
            pltpu.sync_copy(hs_h.at[pl.ds(hoff + HHALF, HHALF)], h_idx_b)
            cps = [
                pltpu.async_copy(emb_h.at[s_idx], s_rows, sem),
                pltpu.async_copy(emb_h.at[t_idx], t_rows, sem),
                pltpu.async_copy(emb_h.at[h_idx_a],
                                 h_rows.at[pl.ds(0, HHALF)], sem),
                pltpu.async_copy(emb_h.at[h_idx_b],
                                 h_rows.at[pl.ds(HHALF, HHALF)], sem),
                pltpu.async_copy(dlt_h.at[s_idx],
                                 dlt_v.at[pl.ds(0, CH)], sem),
            ]
            for cp in cps:
                cp.wait()

            def elem_body(e, carry2):
                svec = [s_rows[e, pl.ds(16 * kk, 16)] for kk in range(8)]

                def dist(row_ref, ridx):
                    acc = jnp.zeros((16,), jnp.float32)
                    for kk in range(8):
                        dd = svec[kk] - row_ref[ridx, pl.ds(16 * kk, 16)]
                        acc = acc + dd * dd
                    return -jnp.sum(acc)

                a0 = jnp.zeros((16,), jnp.float32)
                a1 = jnp.zeros((16,), jnp.float32)
                n0 = jnp.zeros((16,), jnp.float32)
                n1 = jnp.zeros((16,), jnp.float32)
                for h in range(H):
                    dv = dist(h_rows, e * H + h)
                    nv = dist(neg_rows, h)
                    if h < 16:
                        a0 = jnp.where(lanes == h, dv, a0)
                        n0 = jnp.where(lanes == h, nv, n0)
                    else:
                        a1 = jnp.where(lanes == (h - 16), dv, a1)
                        n1 = jnp.where(lanes == (h - 16), nv, n1)
                pmu = dist(t_rows, e)
                a1 = jnp.where(lanes == (H - 16), pmu, a1)
                dvec = dlt_v[pl.ds(0, 16)]
                dl = jnp.sum(jnp.where(lanes == e, dvec, 0.0))
                a1 = jnp.where(lanes == (H - 16 + 1), dl, a1)
                a_v[e, pl.ds(0, 16)] = a0
                a_v[e, pl.ds(16, 16)] = a1
                n_v[e, pl.ds(0, 16)] = n0
                n_v[e, pl.ds(16, 16)] = n1
                return carry2

            lax.fori_loop(0, CH, elem_body, 0)
            pltpu.sync_copy(a_v, a_out.at[pl.ds(off, CH)])
            pltpu.sync_copy(n_v, n_out.at[pl.ds(off, CH)])
            return carry

        lax.fori_loop(0, NCHUNK, chunk_body, 0)

    return k(source, target, h_s_flat, nt, embeddings, delta1d)


def _tc_finish(a_pk, n_pk, times2, h_s_times, h_s_mask):
    BLK = 2048

    def body(a_ref, n_ref, t_ref, ht_ref, hm_ref, o_ref):
        a_full = a_ref[...]
        alpha = a_full[:, :H]
        pmu = a_full[:, H:H + 1]
        dlt = a_full[:, H + 1:H + 2]
        nmu = n_ref[...][:, :H]
        m = jnp.max(alpha, axis=1, keepdims=True)
        ex = jnp.exp(alpha - m)
        attn = ex / jnp.sum(ex, axis=1, keepdims=True)
        d_time = t_ref[...] - ht_ref[...]
        dec = jnp.exp(-dlt * d_time)
        p_lam = pmu + jnp.sum(attn * alpha * dec * hm_ref[...],
                              axis=1, keepdims=True)
        n_lam = jnp.sum(attn * nmu * dec, axis=1, keepdims=True)
        o_ref[...] = -jax.nn.log_sigmoid(p_lam) - jax.nn.log_sigmoid(-n_lam)

    grid = (B // BLK,)
    return pl.pallas_call(
        body,
        grid=grid,
        in_specs=[pl.BlockSpec((BLK, 32), lambda i: (i, 0)),
                  pl.BlockSpec((BLK, 32), lambda i: (i, 0)),
                  pl.BlockSpec((BLK, 1), lambda i: (i, 0)),
                  pl.BlockSpec((BLK, H), lambda i: (i, 0)),
                  pl.BlockSpec((BLK, H), lambda i: (i, 0))],
        out_specs=pl.BlockSpec((BLK, 1), lambda i: (i, 0)),
        out_shape=jax.ShapeDtypeStruct((B, 1), jnp.float32),
    )(a_pk, n_pk, times2, h_s_times, h_s_mask)


def kernel(source, target, times, h_s, h_s_times, h_s_mask, nt,
           embeddings, delta_table):
    h_s_flat = h_s.reshape(-1).astype(jnp.int32)
    a_pk, n_pk = _sc_scores(source.astype(jnp.int32),
                            target.astype(jnp.int32),
                            h_s_flat, nt.astype(jnp.int32),
                            embeddings, delta_table.reshape(-1))
    out2 = _tc_finish(a_pk, n_pk, times[:, None], h_s_times, h_s_mask)
    return out2.reshape(B)


# speedup vs baseline: 5.1003x; 1.3023x over previous
import functools

import jax
import jax.numpy as jnp
from jax import lax
from jax.experimental import pallas as pl
from jax.experimental.pallas import tpu as pltpu
from jax.experimental.pallas import tpu_sc as plsc

NODE = 100000
D = 128
B = 16384
H = 20
NEG = 20

NC = 2
NS = 16
NW = NC * NS
BPW = B // NW
CH = 8
NCHUNK = BPW // CH
HHALF = CH * H // 2


def _sc_scores(source, target, h_s_flat, nt, embeddings, delta1d):
    mesh = plsc.VectorSubcoreMesh(
        core_axis_name="c", subcore_axis_name="s",
        num_cores=NC, num_subcores=NS)

    bufset = [
        pltpu.VMEM((CH,), jnp.int32),
        pltpu.VMEM((CH,), jnp.int32),
        pltpu.VMEM((HHALF,), jnp.int32),
        pltpu.VMEM((HHALF,), jnp.int32),
        pltpu.VMEM((CH, D), jnp.float32),
        pltpu.VMEM((CH, D), jnp.float32),
        pltpu.VMEM((CH * H, D), jnp.float32),
        pltpu.VMEM((16,), jnp.float32),
    ]

    @functools.partial(
        pl.kernel,
        out_type=(jax.ShapeDtypeStruct((B, 32), jnp.float32),
                  jax.ShapeDtypeStruct((B, 32), jnp.float32)),
        mesh=mesh,
        compiler_params=pltpu.CompilerParams(needs_layout_passes=False),
        scratch_types=[
            pltpu.VMEM((NEG,), jnp.int32),
            pltpu.VMEM((NEG, D), jnp.float32),
            *bufset, *bufset,
            pltpu.VMEM((CH, 32), jnp.float32),
            pltpu.VMEM((CH, 32), jnp.float32),
            pltpu.SemaphoreType.DMA,
            pltpu.SemaphoreType.DMA,
        ],
    )
    def k(src_h, tgt_h, hs_h, nt_h, emb_h, dlt_h, a_out, n_out, *scr):
        nt_idx, neg_rows = scr[0], scr[1]
        sets = (scr[2:10], scr[10:18])
        a_v, n_v = scr[18], scr[19]
        sems = (scr[20], scr[21])
        wid = lax.axis_index("s") * NC + lax.axis_index("c")
        base = wid * BPW
        pltpu.sync_copy(nt_h, nt_idx)
        pltpu.async_copy(emb_h.at[nt_idx], neg_rows, sems[0]).wait()
        lanes = lax.iota(jnp.int32, 16)

        def issue(c, bs):
            s_idx, t_idx, h_idx_a, h_idx_b, s_rows, t_rows, h_rows, dlt_v \
                = sets[bs]
            off = base + c * CH
            pltpu.sync_copy(src_h.at[pl.ds(off, CH)], s_idx)
            pltpu.sync_copy(tgt_h.at[pl.ds(off, CH)], t_idx)
            hoff = off * H
            pltpu.sync_copy(hs_h.at[pl.ds(hoff, HHALF)], h_idx_a)
            pltpu.sync_copy(hs_h.at[pl.ds(hoff + HHALF, HHALF)], h_idx_b)
            pltpu.async_copy(emb_h.at[s_idx], s_rows, sems[bs])
            pltpu.async_copy(emb_h.at[t_idx], t_rows, sems[bs])
            pltpu.async_copy(emb_h.at[h_idx_a],
                             h_rows.at[pl.ds(0, HHALF)], sems[bs])
            pltpu.async_copy(emb_h.at[h_idx_b],
                             h_rows.at[pl.ds(HHALF, HHALF)], sems[bs])
            pltpu.async_copy(dlt_h.at[s_idx], dlt_v.at[pl.ds(0, CH)],
                             sems[bs])

        def drain(bs):
            s_idx, t_idx, h_idx_a, h_idx_b, s_rows, t_rows, h_rows, dlt_v \
                = sets[bs]
            pltpu.make_async_copy(emb_h.at[s_idx], s_rows, sems[bs]).wait()
            pltpu.make_async_copy(emb_h.at[t_idx], t_rows, sems[bs]).wait()
            pltpu.make_async_copy(emb_h.at[h_idx_a],
                                  h_rows.at[pl.ds(0, HHALF)],
                                  sems[bs]).wait()
            pltpu.make_async_copy(emb_h.at[h_idx_b],
                                  h_rows.at[pl.ds(HHALF, HHALF)],
                                  sems[bs]).wait()
            pltpu.make_async_copy(dlt_h.at[s_idx], dlt_v.at[pl.ds(0, CH)],
                                  sems[bs]).wait()

        def compute(c, bs):
            _, _, _, _, s_rows, t_rows, h_rows, dlt_v = sets[bs]
            off = base + c * CH

            def elem_body(e, carry2):
                svec = [s_rows[e, pl.ds(16 * kk, 16)] for kk in range(8)]

                def dist(row_ref, ridx):
                    acc = jnp.zeros((16,), jnp.float32)
                    for kk in range(8):
                        dd = svec[kk] - row_ref[ridx, pl.ds(16 * kk, 16)]
                        acc = acc + dd * dd
                    return -jnp.sum(acc)

                a0 = jnp.zeros((16,), jnp.float32)
                a1 = jnp.zeros((16,), jnp.float32)
                n0 = jnp.zeros((16,), jnp.float32)
                n1 = jnp.zeros((16,), jnp.float32)
                for h in range(H):
                    dv = dist(h_rows, e * H + h)
                    nv = dist(neg_rows, h)
                    if h < 16:
                        a0 = jnp.where(lanes == h, dv, a0)
                        n0 = jnp.where(lanes == h, nv, n0)
                    else:
                        a1 = jnp.where(lanes == (h - 16), dv, a1)
                        n1 = jnp.where(lanes == (h - 16), nv, n1)
                pmu = dist(t_rows, e)
                a1 = jnp.where(lanes == (H - 16), pmu, a1)
                dvec = dlt_v[pl.ds(0, 16)]
                dl = jnp.sum(jnp.where(lanes == e, dvec, 0.0))
                a1 = jnp.where(lanes == (H - 16 + 1), dl, a1)
                a_v[e, pl.ds(0, 16)] = a0
                a_v[e, pl.ds(16, 16)] = a1
                n_v[e, pl.ds(0, 16)] = n0
                n_v[e, pl.ds(16, 16)] = n1
                return carry2

            lax.fori_loop(0, CH, elem_body, 0)
            pltpu.sync_copy(a_v, a_out.at[pl.ds(off, CH)])
            pltpu.sync_copy(n_v, n_out.at[pl.ds(off, CH)])

        issue(0, 0)
        issue(1, 1)

        def pair_body(g, carry):
            for b2 in range(2):
                c = 2 * g + b2
                drain(b2)
                compute(c, b2)
                nxt = c + 2

                @pl.when(nxt < NCHUNK)
                def _():
                    issue(nxt, b2)
            return carry

        lax.fori_loop(0, NCHUNK // 2, pair_body, 0)

    return k(source, target, h_s_flat, nt, embeddings, delta1d)


def _tc_finish(a_pk, n_pk, times2, h_s_times, h_s_mask):
    BLK = 2048

    def body(a_ref, n_ref, t_ref, ht_ref, hm_ref, o_ref):
        a_full = a_ref[...]
        alpha = a_full[:, :H]
        pmu = a_full[:, H:H + 1]
        dlt = a_full[:, H + 1:H + 2]
        nmu = n_ref[...][:, :H]
        m = jnp.max(alpha, axis=1, keepdims=True)
        ex = jnp.exp(alpha - m)
        attn = ex / jnp.sum(ex, axis=1, keepdims=True)
        d_time = t_ref[...] - ht_ref[...]
        dec = jnp.exp(-dlt * d_time)
        p_lam = pmu + jnp.sum(attn * alpha * dec * hm_ref[...],
                              axis=1, keepdims=True)
        n_lam = jnp.sum(attn * nmu * dec, axis=1, keepdims=True)
        o_ref[...] = -jax.nn.log_sigmoid(p_lam) - jax.nn.log_sigmoid(-n_lam)

    grid = (B // BLK,)
    return pl.pallas_call(
        body,
        grid=grid,
        in_specs=[pl.BlockSpec((BLK, 32), lambda i: (i, 0)),
                  pl.BlockSpec((BLK, 32), lambda i: (i, 0)),
                  pl.BlockSpec((BLK, 1), lambda i: (i, 0)),
                  pl.BlockSpec((BLK, H), lambda i: (i, 0)),
                  pl.BlockSpec((BLK, H), lambda i: (i, 0))],
        out_specs=pl.BlockSpec((BLK, 1), lambda i: (i, 0)),
        out_shape=jax.ShapeDtypeStruct((B, 1), jnp.float32),
    )(a_pk, n_pk, times2, h_s_times, h_s_mask)


def kernel(source, target, times, h_s, h_s_times, h_s_mask, nt,
           embeddings, delta_table):
    h_s_flat = h_s.reshape(-1).astype(jnp.int32)
    a_pk, n_pk = _sc_scores(source.astype(jnp.int32),
                            target.astype(jnp.int32),
                            h_s_flat, nt.astype(jnp.int32),
                            embeddings, delta_table.reshape(-1))
    out2 = _tc_finish(a_pk, n_pk, times[:, None], h_s_times, h_s_mask)
    return out2.reshape(B)
